# Initial kernel scaffold; baseline (speedup 1.0000x reference)
import jax
import jax.numpy as jnp
from jax.experimental import pallas as pl


def kernel(x, batch, edge_index, edge_type):
    raise NotImplementedError("write your pallas kernel here")


# speedup vs baseline: 2.0019x; 2.0019x over previous
import functools

import jax
import jax.numpy as jnp
from jax import lax
from jax.experimental import pallas as pl
from jax.experimental.pallas import tpu as pltpu
from jax.experimental.pallas import tpu_sc as plsc

N_NODES = 10000
E_EDGES = 320000
N_GRAPHS = 128
N_CHUNKS = 32
CHUNK = E_EDGES // N_CHUNKS
LSUB = CHUNK // 16
ROWS = 125
COLS = 80

_mesh = plsc.VectorSubcoreMesh(core_axis_name="c", subcore_axis_name="s")


@functools.partial(
    pl.kernel,
    out_type=[
        jax.ShapeDtypeStruct((2 * E_EDGES,), jnp.int32),
        jax.ShapeDtypeStruct((E_EDGES,), jnp.int32),
    ],
    mesh=_mesh,
    compiler_params=pltpu.CompilerParams(needs_layout_passes=False),
    scratch_types=[
        pltpu.VMEM((N_NODES,), jnp.int32),
        pltpu.VMEM((ROWS, COLS), jnp.int32),
        pltpu.VMEM((ROWS, COLS), jnp.int32),
        pltpu.VMEM((ROWS, COLS), jnp.int32),
        pltpu.VMEM((ROWS, COLS), jnp.int32),
        pltpu.VMEM((ROWS, COLS), jnp.int32),
        pltpu.VMEM((ROWS, COLS), jnp.int32),
        pltpu.VMEM((16, N_GRAPHS), jnp.int32),
        pltpu.VMEM((16, N_GRAPHS), jnp.int32),
        pltpu.VMEM((N_GRAPHS,), jnp.int32),
        pltpu.VMEM((N_CHUNKS, N_GRAPHS), jnp.int32),
        pltpu.VMEM_SHARED((N_CHUNKS, 16, N_GRAPHS), jnp.int32),
        pltpu.VMEM_SHARED((N_CHUNKS, N_GRAPHS), jnp.int32),
        pltpu.SemaphoreType.DMA,
    ],
)
def _sort_edges(batch_hbm, nin_hbm, nout_hbm, et_hbm, ei_out, et_out,
                batch_v, nin_a, nin_b, nout_a, et_a, dest_lo, dest_hi,
                cnt, cur, ct_v, ct_all, sh_cnt, sh_ct, sem):
    c = lax.axis_index("c")
    s = lax.axis_index("s")
    ka = c * 16 + s
    kb = (1 - c) * 16 + s
    lane = lax.iota(jnp.int32, 16)
    l625 = lane * LSUB
    zeros16 = jnp.zeros((16,), jnp.int32)

    pltpu.sync_copy(batch_hbm, batch_v)
    pltpu.sync_copy(nin_hbm.at[ka], nin_a)
    pltpu.sync_copy(nin_hbm.at[kb], nin_b)
    pltpu.sync_copy(nout_hbm.at[ka], nout_a)
    pltpu.sync_copy(et_hbm.at[ka], et_a)

    def zero_cnt():
        for l in range(16):
            for gc in range(N_GRAPHS // 16):
                cnt[l, pl.ds(gc * 16, 16)] = zeros16

    def count_chunk(nin_ref):
        def body(t, carry):
            p = l625 + t
            row = p // COLS
            col = p - row * COLS
            nin_v = plsc.load_gather(nin_ref, [row, col])
            g = plsc.load_gather(batch_v, [nin_v])
            cvals = plsc.load_gather(cnt, [lane, g])
            plsc.store_scatter(cnt, [lane, g], cvals + 1)
            return carry
        lax.fori_loop(0, LSUB, body, 0)

    def publish(k):
        pltpu.sync_copy(cnt, sh_cnt.at[k])
        for gc in range(N_GRAPHS // 16):
            ssum = zeros16
            for l in range(16):
                ssum = ssum + cnt[l, pl.ds(gc * 16, 16)]
            ct_v[pl.ds(gc * 16, 16)] = ssum
        pltpu.sync_copy(ct_v, sh_ct.at[k])

    zero_cnt()
    count_chunk(nin_a)
    publish(ka)
    zero_cnt()
    count_chunk(nin_b)
    publish(kb)

    plsc.subcore_barrier()

    pltpu.sync_copy(sh_ct, ct_all)
    pltpu.sync_copy(sh_cnt.at[ka], cnt)
    carry = jnp.int32(0)
    for gc in range(N_GRAPHS // 16):
        total = zeros16
        pre = zeros16
        for k in range(N_CHUNKS):
            v = ct_all[k, pl.ds(gc * 16, 16)]
            total = total + v
            pre = pre + jnp.where(k < ka, v, zeros16)
        run = plsc.cumsum(total) - total + carry + pre
        carry = carry + jnp.sum(total)
        for l in range(16):
            cur[l, pl.ds(gc * 16, 16)] = run
            run = run + cnt[l, pl.ds(gc * 16, 16)]

    def place_body(t, carry2):
        p = l625 + t
        row = p // COLS
        col = p - row * COLS
        nin_v = plsc.load_gather(nin_a, [row, col])
        g = plsc.load_gather(batch_v, [nin_v])
        d = plsc.load_gather(cur, [lane, g])
        plsc.store_scatter(cur, [lane, g], d + 1)
        plsc.store_scatter(dest_lo, [row, col], d)
        plsc.store_scatter(dest_hi, [row, col], d + E_EDGES)
        return carry2
    lax.fori_loop(0, LSUB, place_body, 0)

    GB = 5
    def scatter_group(jg, carry3):
        copies = []
        for jj in range(GB):
            j = jg * GB + jj
            copies.append(pltpu.async_copy(nin_a.at[j], ei_out.at[dest_lo.at[j]], sem))
            copies.append(pltpu.async_copy(nout_a.at[j], ei_out.at[dest_hi.at[j]], sem))
            copies.append(pltpu.async_copy(et_a.at[j], et_out.at[dest_lo.at[j]], sem))
        for cp in copies:
            cp.wait()
        return carry3
    lax.fori_loop(0, ROWS // GB, scatter_group, 0)


def kernel(x, batch, edge_index, edge_type):
    nin3 = edge_index[0].reshape(N_CHUNKS, ROWS, COLS).astype(jnp.int32)
    nout3 = edge_index[1].reshape(N_CHUNKS, ROWS, COLS).astype(jnp.int32)
    et3 = edge_type.reshape(N_CHUNKS, ROWS, COLS).astype(jnp.int32)
    ei_flat, et_sorted = _sort_edges(batch.astype(jnp.int32), nin3, nout3, et3)
    edge_index_sorted = ei_flat.reshape(2, E_EDGES)
    edge_weight = jnp.ones((E_EDGES,), x.dtype)
    return x, edge_index_sorted, et_sorted, edge_weight
